# baseline (speedup 1.0000x reference)
import functools

import jax
import jax.numpy as jnp
from jax import lax
from jax.experimental import pallas as pl
from jax.experimental.pallas import tpu as pltpu
from jax.experimental.pallas import tpu_sc as plsc

N = 10000
E = 320000
D = 128
H = D // 2

NC = 2
NS = 16
K = 64
NCHUNK = 320
EPT = NCHUNK * K
E2 = EPT * NS
NBUF = 2

N2 = 10240
RPT = N2 // NS
RCHUNK = 80
NRC = RPT // RCHUNK


def _sc_segment_sum(table, srcr, dstr, zrows):
    mesh = plsc.VectorSubcoreMesh(
        core_axis_name="c", subcore_axis_name="s", num_cores=NC, num_subcores=NS
    )

    @functools.partial(
        pl.kernel,
        out_type=jax.ShapeDtypeStruct((N2, D), jnp.float32),
        mesh=mesh,
        compiler_params=pltpu.CompilerParams(use_tc_tiling_on_sc=False),
        scratch_types=[
            pltpu.VMEM((NCHUNK, K), jnp.int32),
            pltpu.VMEM((NCHUNK, K), jnp.int32),
            *[pltpu.VMEM((K, H), jnp.float32) for _ in range(NBUF)],
            pltpu.VMEM((RCHUNK, H), jnp.float32),
            pltpu.VMEM_SHARED((N2, H), jnp.float32),
            *[pltpu.SemaphoreType.DMA for _ in range(NBUF)],
        ],
    )
    def body(y_hbm, src_hbm, dst_hbm, z_hbm, out_hbm, src_v, dst_v, *rest):
        rows = rest[:NBUF]
        zbuf = rest[NBUF]
        acc = rest[NBUF + 1]
        gsem = rest[NBUF + 2:]
        c = lax.axis_index("c")
        s = lax.axis_index("s")
        col = c * H

        pltpu.sync_copy(src_hbm.at[c, s], src_v)
        pltpu.sync_copy(dst_hbm.at[s], dst_v)

        pltpu.sync_copy(z_hbm, zbuf)
        for q in range(NRC):
            pltpu.sync_copy(zbuf, acc.at[pl.ds(s * RPT + q * RCHUNK, RCHUNK)])
        plsc.subcore_barrier()

        for b in range(NBUF):
            pltpu.async_copy(y_hbm.at[src_v.at[b]], rows[b], gsem[b])

        def ring(i, carry):
            j = NBUF * i
            for b in range(NBUF):
                pltpu.make_async_copy(y_hbm.at[src_v.at[j + b]], rows[b], gsem[b]).wait()
                pltpu.sync_copy(rows[b], acc.at[dst_v.at[j + b]], add=True)
                pltpu.async_copy(y_hbm.at[src_v.at[j + NBUF + b]], rows[b], gsem[b])
            return carry

        lax.fori_loop(0, NCHUNK // NBUF - 1, ring, 0)
        j = NCHUNK - NBUF
        for b in range(NBUF):
            pltpu.make_async_copy(y_hbm.at[src_v.at[j + b]], rows[b], gsem[b]).wait()
            pltpu.sync_copy(rows[b], acc.at[dst_v.at[j + b]], add=True)
        plsc.subcore_barrier()

        for q in range(NRC):
            r = s * RPT + q * RCHUNK
            pltpu.sync_copy(acc.at[pl.ds(r, RCHUNK)], zbuf)
            pltpu.sync_copy(zbuf, out_hbm.at[pl.ds(r, RCHUNK), pl.ds(col, H)])

    return body(table, srcr, dstr, zrows)


_RB = 1024


def _mm_body(x_ref, w_ref, o_ref):
    o_ref[...] = lax.dot_general(
        x_ref[...], w_ref[...], (((1,), (1,)), ((), ())),
        preferred_element_type=jnp.float32,
    )


def _tc_xwt_split(x, w):
    nb = N2 // _RB
    return pl.pallas_call(
        _mm_body,
        grid=(nb, NC),
        in_specs=[
            pl.BlockSpec((_RB, D), lambda i, c: (i, 0)),
            pl.BlockSpec((H, D), lambda i, c: (c, 0)),
        ],
        out_specs=pl.BlockSpec((_RB, H), lambda i, c: (c * nb + i, 0)),
        out_shape=jax.ShapeDtypeStruct((2 * N2, H), jnp.float32),
    )(x, w)


def _relu_mm_body(p_ref, b_ref, w_ref, o_ref):
    h = jnp.maximum(p_ref[...] + b_ref[...], 0.0)
    o_ref[...] = lax.dot_general(
        h, w_ref[...], (((1,), (1,)), ((), ())),
        preferred_element_type=jnp.float32,
    )


def _tc_relu_then_xwt_split(p, b, w):
    nb = N2 // _RB
    return pl.pallas_call(
        _relu_mm_body,
        grid=(nb, NC),
        in_specs=[
            pl.BlockSpec((_RB, D), lambda i, c: (i, 0)),
            pl.BlockSpec((1, D), lambda i, c: (0, 0)),
            pl.BlockSpec((H, D), lambda i, c: (c, 0)),
        ],
        out_specs=pl.BlockSpec((_RB, H), lambda i, c: (c * nb + i, 0)),
        out_shape=jax.ShapeDtypeStruct((2 * N2, H), jnp.float32),
    )(p, b, w)


def _relu_body(p_ref, b_ref, o_ref):
    o_ref[...] = jnp.maximum(p_ref[...] + b_ref[...], 0.0)


def _tc_relu(p, b):
    rb = 2000
    return pl.pallas_call(
        _relu_body,
        grid=(N // rb,),
        in_specs=[
            pl.BlockSpec((rb, D), lambda i: (i, 0)),
            pl.BlockSpec((1, D), lambda i: (0, 0)),
        ],
        out_specs=pl.BlockSpec((rb, D), lambda i: (i, 0)),
        out_shape=jax.ShapeDtypeStruct((N, D), jnp.float32),
    )(p, b)


def kernel(x, edge_index, W1, b1, W2, b2):
    ei = edge_index.astype(jnp.int32)
    srcp = jnp.concatenate([ei[0], jnp.zeros((E2 - E,), jnp.int32)])
    pad_dst = N + jnp.arange(E2 - E, dtype=jnp.int32) % (N2 - N)
    dstp = jnp.concatenate([ei[1], pad_dst])
    core_off = (jnp.arange(NC, dtype=jnp.int32) * N2).reshape(NC, 1, 1, 1)
    srcr = srcp.reshape(1, NS, NCHUNK, K) + core_off
    dstr = dstp.reshape(NS, NCHUNK, K)
    zrows = jnp.zeros((RCHUNK, H), jnp.float32)
    b1r = b1.reshape(1, D)
    b2r = b2.reshape(1, D)
    xp = jnp.pad(x, ((0, N2 - N), (0, 0)))

    t1 = _tc_xwt_split(xp, W1)
    p = _sc_segment_sum(t1, srcr, dstr, zrows)
    t2 = _tc_relu_then_xwt_split(p, b1r, W2)
    q = _sc_segment_sum(t2, srcr, dstr, zrows)
    return _tc_relu(q, b2r)


# speedup vs baseline: 1.9480x; 1.9480x over previous
import functools

import jax
import jax.numpy as jnp
from jax import lax
from jax.experimental import pallas as pl
from jax.experimental.pallas import tpu as pltpu
from jax.experimental.pallas import tpu_sc as plsc

N = 10000
E = 320000
D = 128
H = D // 2

NC = 2
NS = 16
K = 80
NCHUNK = 250
EPT = NCHUNK * K
E2 = EPT * NS
NBUF = 2

N2 = 10240
RPT = N2 // NS
RCHUNK = 80
NRC = RPT // RCHUNK


def _sc_segment_sum(table, srcr, dstr, zrows):
    mesh = plsc.VectorSubcoreMesh(
        core_axis_name="c", subcore_axis_name="s", num_cores=NC, num_subcores=NS
    )

    @functools.partial(
        pl.kernel,
        out_type=jax.ShapeDtypeStruct((N2, D), jnp.float32),
        mesh=mesh,
        compiler_params=pltpu.CompilerParams(use_tc_tiling_on_sc=False),
        scratch_types=[
            pltpu.VMEM((NCHUNK, K), jnp.int32),
            pltpu.VMEM((NCHUNK, K), jnp.int32),
            *[pltpu.VMEM((K, H), jnp.float32) for _ in range(NBUF)],
            pltpu.VMEM((RCHUNK, H), jnp.float32),
            pltpu.VMEM_SHARED((N2, H), jnp.float32),
            *[pltpu.SemaphoreType.DMA for _ in range(NBUF)],
        ],
    )
    def body(y_hbm, src_hbm, dst_hbm, z_hbm, out_hbm, src_v, dst_v, *rest):
        rows = rest[:NBUF]
        zbuf = rest[NBUF]
        acc = rest[NBUF + 1]
        gsem = rest[NBUF + 2:]
        c = lax.axis_index("c")
        s = lax.axis_index("s")
        col = c * H

        pltpu.sync_copy(src_hbm.at[c, s], src_v)
        pltpu.sync_copy(dst_hbm.at[s], dst_v)

        pltpu.sync_copy(z_hbm, zbuf)
        for q in range(NRC):
            pltpu.sync_copy(zbuf, acc.at[pl.ds(s * RPT + q * RCHUNK, RCHUNK)])
        plsc.subcore_barrier()

        for b in range(NBUF):
            pltpu.async_copy(y_hbm.at[src_v.at[b]], rows[b], gsem[b])

        def ring(i, carry):
            j = NBUF * i
            for b in range(NBUF):
                pltpu.make_async_copy(y_hbm.at[src_v.at[j + b]], rows[b], gsem[b]).wait()
                pltpu.sync_copy(rows[b], acc.at[dst_v.at[j + b]], add=True)
                pltpu.async_copy(y_hbm.at[src_v.at[j + NBUF + b]], rows[b], gsem[b])
            return carry

        lax.fori_loop(0, NCHUNK // NBUF - 1, ring, 0)
        j = NCHUNK - NBUF
        for b in range(NBUF):
            pltpu.make_async_copy(y_hbm.at[src_v.at[j + b]], rows[b], gsem[b]).wait()
            pltpu.sync_copy(rows[b], acc.at[dst_v.at[j + b]], add=True)
        plsc.subcore_barrier()

        for q in range(NRC):
            r = s * RPT + q * RCHUNK
            pltpu.sync_copy(acc.at[pl.ds(r, RCHUNK)], zbuf)
            pltpu.sync_copy(zbuf, out_hbm.at[pl.ds(r, RCHUNK), pl.ds(col, H)])

    return body(table, srcr, dstr, zrows)


_RB = 1024


def _mm_body(x_ref, w_ref, o_ref):
    o_ref[...] = lax.dot_general(
        x_ref[...], w_ref[...], (((1,), (1,)), ((), ())),
        preferred_element_type=jnp.float32,
    )


def _tc_xwt_split(x, w):
    nb = N2 // _RB
    return pl.pallas_call(
        _mm_body,
        grid=(nb, NC),
        in_specs=[
            pl.BlockSpec((_RB, D), lambda i, c: (i, 0)),
            pl.BlockSpec((H, D), lambda i, c: (c, 0)),
        ],
        out_specs=pl.BlockSpec((_RB, H), lambda i, c: (c * nb + i, 0)),
        out_shape=jax.ShapeDtypeStruct((2 * N2, H), jnp.float32),
    )(x, w)


def _relu_mm_body(p_ref, b_ref, w_ref, o_ref):
    h = jnp.maximum(p_ref[...] + b_ref[...], 0.0)
    o_ref[...] = lax.dot_general(
        h, w_ref[...], (((1,), (1,)), ((), ())),
        preferred_element_type=jnp.float32,
    )


def _tc_relu_then_xwt_split(p, b, w):
    nb = N2 // _RB
    return pl.pallas_call(
        _relu_mm_body,
        grid=(nb, NC),
        in_specs=[
            pl.BlockSpec((_RB, D), lambda i, c: (i, 0)),
            pl.BlockSpec((1, D), lambda i, c: (0, 0)),
            pl.BlockSpec((H, D), lambda i, c: (c, 0)),
        ],
        out_specs=pl.BlockSpec((_RB, H), lambda i, c: (c * nb + i, 0)),
        out_shape=jax.ShapeDtypeStruct((2 * N2, H), jnp.float32),
    )(p, b, w)


def _relu_body(p_ref, b_ref, o_ref):
    o_ref[...] = jnp.maximum(p_ref[...] + b_ref[...], 0.0)


def _tc_relu(p, b):
    rb = 2000
    return pl.pallas_call(
        _relu_body,
        grid=(N // rb,),
        in_specs=[
            pl.BlockSpec((rb, D), lambda i: (i, 0)),
            pl.BlockSpec((1, D), lambda i: (0, 0)),
        ],
        out_specs=pl.BlockSpec((rb, D), lambda i: (i, 0)),
        out_shape=jax.ShapeDtypeStruct((N, D), jnp.float32),
    )(p, b)


def kernel(x, edge_index, W1, b1, W2, b2):
    ei = edge_index.astype(jnp.int32)
    srcp = jnp.concatenate([ei[0], jnp.zeros((E2 - E,), jnp.int32)])
    pad_dst = N + jnp.arange(E2 - E, dtype=jnp.int32) % (N2 - N)
    dstp = jnp.concatenate([ei[1], pad_dst])
    core_off = (jnp.arange(NC, dtype=jnp.int32) * N2).reshape(NC, 1, 1, 1)
    srcr = srcp.reshape(1, NS, NCHUNK, K) + core_off
    dstr = dstp.reshape(NS, NCHUNK, K)
    zrows = jnp.zeros((RCHUNK, H), jnp.float32)
    b1r = b1.reshape(1, D)
    b2r = b2.reshape(1, D)
    xp = jnp.pad(x, ((0, N2 - N), (0, 0)))

    t1 = _tc_xwt_split(xp, W1)
    p = _sc_segment_sum(t1, srcr, dstr, zrows)
    t2 = _tc_relu_then_xwt_split(p, b1r, W2)
    q = _sc_segment_sum(t2, srcr, dstr, zrows)
    return _tc_relu(q, b2r)
